# baseline (speedup 1.0000x reference)
import jax
import jax.numpy as jnp
from jax.experimental import pallas as pl
from jax.experimental.pallas import tpu as pltpu

_BLOCK_ROWS = 2560
_K_SPLIT = 2

_DN = (((1,), (1,)), ((), ()))


def _fused_linear_kernel(x_ref, wc_ref, bc_ref, wb_ref, bb_ref,
                         scores_t_ref, deltas_t_ref):
    k = pl.program_id(1)
    kc = x_ref.shape[1]
    x = x_ref[...]
    ks = pl.ds(k * kc, kc)
    sc = jax.lax.dot_general(wc_ref[:, ks], x, _DN,
                             preferred_element_type=jnp.float32)
    dl = jax.lax.dot_general(wb_ref[:, ks], x, _DN,
                             preferred_element_type=jnp.float32)

    @pl.when(k == 0)
    def _init():
        scores_t_ref[...] = sc + bc_ref[...].T
        deltas_t_ref[...] = dl + bb_ref[...].T

    @pl.when(k > 0)
    def _acc():
        scores_t_ref[...] += sc
        deltas_t_ref[...] += dl


@jax.jit
def kernel(x, W_cls, b_cls, W_box, b_box):
    if x.ndim > 2:
        x = x.reshape(x.shape[0], -1)
    n, in_dim = x.shape
    n_cls = W_cls.shape[0]
    n_box = W_box.shape[0]
    kc = in_dim // _K_SPLIT

    bc = b_cls.reshape(1, n_cls)
    bb = b_box.reshape(1, n_box)

    grid = (pl.cdiv(n, _BLOCK_ROWS), _K_SPLIT)
    scores_t, deltas_t = pl.pallas_call(
        _fused_linear_kernel,
        grid=grid,
        in_specs=[
            pl.BlockSpec((_BLOCK_ROWS, kc), lambda i, k: (i, k)),
            pl.BlockSpec((n_cls, in_dim), lambda i, k: (0, 0)),
            pl.BlockSpec((1, n_cls), lambda i, k: (0, 0)),
            pl.BlockSpec((n_box, in_dim), lambda i, k: (0, 0)),
            pl.BlockSpec((1, n_box), lambda i, k: (0, 0)),
        ],
        out_specs=[
            pl.BlockSpec((n_cls, _BLOCK_ROWS), lambda i, k: (0, i)),
            pl.BlockSpec((n_box, _BLOCK_ROWS), lambda i, k: (0, i)),
        ],
        out_shape=[
            jax.ShapeDtypeStruct((n_cls, n), jnp.float32),
            jax.ShapeDtypeStruct((n_box, n), jnp.float32),
        ],
        compiler_params=pltpu.CompilerParams(
            dimension_semantics=("arbitrary", "arbitrary"),
        ),
    )(x, W_cls, bc, W_box, bb)
    return (scores_t.T, deltas_t.T)


# speedup vs baseline: 1.2444x; 1.2444x over previous
import jax
import jax.numpy as jnp
from jax.experimental import pallas as pl
from jax.experimental.pallas import tpu as pltpu

_BLOCK_ROWS = 2560
_HALF = _BLOCK_ROWS // 2

_DN = (((1,), (1,)), ((), ()))


def _fused_linear_kernel(xa_ref, xb_ref, wc_ref, bc_ref, wb_ref, bb_ref,
                         scores_t_ref, deltas_t_ref):
    xa = xa_ref[...]
    xb = xb_ref[...]
    bc = bc_ref[...].T
    bb = bb_ref[...].T
    scores_t_ref[:, :_HALF] = (
        jax.lax.dot_general(wc_ref[...], xa, _DN,
                            preferred_element_type=jnp.float32) + bc
    )
    deltas_t_ref[:, :_HALF] = (
        jax.lax.dot_general(wb_ref[...], xa, _DN,
                            preferred_element_type=jnp.float32) + bb
    )
    scores_t_ref[:, _HALF:] = (
        jax.lax.dot_general(wc_ref[...], xb, _DN,
                            preferred_element_type=jnp.float32) + bc
    )
    deltas_t_ref[:, _HALF:] = (
        jax.lax.dot_general(wb_ref[...], xb, _DN,
                            preferred_element_type=jnp.float32) + bb
    )


@jax.jit
def kernel(x, W_cls, b_cls, W_box, b_box):
    if x.ndim > 2:
        x = x.reshape(x.shape[0], -1)
    n, in_dim = x.shape
    n_cls = W_cls.shape[0]
    n_box = W_box.shape[0]

    bc = b_cls.reshape(1, n_cls)
    bb = b_box.reshape(1, n_box)

    grid = (pl.cdiv(n, _BLOCK_ROWS),)
    scores_t, deltas_t = pl.pallas_call(
        _fused_linear_kernel,
        grid=grid,
        in_specs=[
            pl.BlockSpec((_HALF, in_dim), lambda i: (2 * i, 0)),
            pl.BlockSpec((_HALF, in_dim), lambda i: (2 * i + 1, 0)),
            pl.BlockSpec((n_cls, in_dim), lambda i: (0, 0)),
            pl.BlockSpec((1, n_cls), lambda i: (0, 0)),
            pl.BlockSpec((n_box, in_dim), lambda i: (0, 0)),
            pl.BlockSpec((1, n_box), lambda i: (0, 0)),
        ],
        out_specs=[
            pl.BlockSpec((n_cls, _BLOCK_ROWS), lambda i: (0, i)),
            pl.BlockSpec((n_box, _BLOCK_ROWS), lambda i: (0, i)),
        ],
        out_shape=[
            jax.ShapeDtypeStruct((n_cls, n), jnp.float32),
            jax.ShapeDtypeStruct((n_box, n), jnp.float32),
        ],
        compiler_params=pltpu.CompilerParams(
            dimension_semantics=("arbitrary",),
        ),
    )(x, x, W_cls, bc, W_box, bb)
    return (scores_t.T, deltas_t.T)


# speedup vs baseline: 1.3417x; 1.0782x over previous
import jax
import jax.numpy as jnp
from jax.experimental import pallas as pl
from jax.experimental.pallas import tpu as pltpu

_BR = 2560
_N = 20000
_NB = (_N + _BR - 1) // _BR
_R_LAST = _N - (_NB - 1) * _BR
_DN = (((1,), (1,)), ((), ()))


def _kernel(x_hbm, wc_ref, bc_ref, wb_ref, bb_ref, st_hbm, dt_hbm,
            xb0, xb1, sb0, sb1, db0, db1, sbl, dbl, in_sem, os_sem, od_sem):
    xbufs = (xb0, xb1)
    sbufs = (sb0, sb1)
    dbufs = (db0, db1)

    def in_copy(i):
        r = _BR if i < _NB - 1 else _R_LAST
        return pltpu.make_async_copy(
            x_hbm.at[pl.ds(i * _BR, r), :],
            xbufs[i % 2].at[pl.ds(0, r), :],
            in_sem.at[i % 2],
        )

    def s_copy(i):
        if i < _NB - 1:
            return pltpu.make_async_copy(
                sbufs[i % 2], st_hbm.at[:, pl.ds(i * _BR, _BR)],
                os_sem.at[i % 2])
        return pltpu.make_async_copy(
            sbl, st_hbm.at[:, pl.ds(i * _BR, _R_LAST)], os_sem.at[i % 2])

    def d_copy(i):
        if i < _NB - 1:
            return pltpu.make_async_copy(
                dbufs[i % 2], dt_hbm.at[:, pl.ds(i * _BR, _BR)],
                od_sem.at[i % 2])
        return pltpu.make_async_copy(
            dbl, dt_hbm.at[:, pl.ds(i * _BR, _R_LAST)], od_sem.at[i % 2])

    bc = bc_ref[...].T
    bb = bb_ref[...].T
    wc = wc_ref[...]
    wb = wb_ref[...]

    in_copy(0).start()
    for i in range(_NB):
        if i + 1 < _NB:
            in_copy(i + 1).start()
        in_copy(i).wait()
        if i >= 2:
            s_copy(i - 2).wait()
            d_copy(i - 2).wait()
        s = i % 2
        if i < _NB - 1:
            x = xbufs[s][...]
            sbufs[s][...] = jax.lax.dot_general(
                wc, x, _DN, preferred_element_type=jnp.float32) + bc
            dbufs[s][...] = jax.lax.dot_general(
                wb, x, _DN, preferred_element_type=jnp.float32) + bb
        else:
            x = xbufs[s][pl.ds(0, _R_LAST), :]
            sbl[...] = jax.lax.dot_general(
                wc, x, _DN, preferred_element_type=jnp.float32) + bc
            dbl[...] = jax.lax.dot_general(
                wb, x, _DN, preferred_element_type=jnp.float32) + bb
        s_copy(i).start()
        d_copy(i).start()
    for i in (_NB - 2, _NB - 1):
        s_copy(i).wait()
        d_copy(i).wait()


@jax.jit
def kernel(x, W_cls, b_cls, W_box, b_box):
    if x.ndim > 2:
        x = x.reshape(x.shape[0], -1)
    n, in_dim = x.shape
    n_cls = W_cls.shape[0]
    n_box = W_box.shape[0]

    bc = b_cls.reshape(1, n_cls)
    bb = b_box.reshape(1, n_box)

    scores_t, deltas_t = pl.pallas_call(
        _kernel,
        in_specs=[
            pl.BlockSpec(memory_space=pltpu.HBM),
            pl.BlockSpec(memory_space=pltpu.VMEM),
            pl.BlockSpec(memory_space=pltpu.VMEM),
            pl.BlockSpec(memory_space=pltpu.VMEM),
            pl.BlockSpec(memory_space=pltpu.VMEM),
        ],
        out_specs=[
            pl.BlockSpec(memory_space=pltpu.HBM),
            pl.BlockSpec(memory_space=pltpu.HBM),
        ],
        out_shape=[
            jax.ShapeDtypeStruct((n_cls, n), jnp.float32),
            jax.ShapeDtypeStruct((n_box, n), jnp.float32),
        ],
        scratch_shapes=[
            pltpu.VMEM((_BR, in_dim), jnp.float32),
            pltpu.VMEM((_BR, in_dim), jnp.float32),
            pltpu.VMEM((n_cls, _BR), jnp.float32),
            pltpu.VMEM((n_cls, _BR), jnp.float32),
            pltpu.VMEM((n_box, _BR), jnp.float32),
            pltpu.VMEM((n_box, _BR), jnp.float32),
            pltpu.VMEM((n_cls, _R_LAST), jnp.float32),
            pltpu.VMEM((n_box, _R_LAST), jnp.float32),
            pltpu.SemaphoreType.DMA((2,)),
            pltpu.SemaphoreType.DMA((2,)),
            pltpu.SemaphoreType.DMA((2,)),
        ],
    )(x, W_cls, bc, W_box, bb)
    return (scores_t.T, deltas_t.T)


# speedup vs baseline: 1.3573x; 1.0116x over previous
import jax
import jax.numpy as jnp
from jax.experimental import pallas as pl
from jax.experimental.pallas import tpu as pltpu

_BR = 2560
_N = 20000
_NB = (_N + _BR - 1) // _BR
_R_LAST = _N - (_NB - 1) * _BR
_DN = (((1,), (1,)), ((), ()))


def _kernel(x_hbm, wc_ref, bc_ref, wb_ref, bb_ref, st_hbm, dt_hbm,
            xb0, xb1, sb0, sb1, db0, db1, sbl, dbl, in_sem, os_sem, od_sem):
    xbufs = (xb0, xb1)
    sbufs = (sb0, sb1)
    dbufs = (db0, db1)

    def in_copies(i):
        r = _BR if i < _NB - 1 else _R_LAST
        h = r // 2
        return (
            pltpu.make_async_copy(
                x_hbm.at[pl.ds(i * _BR, h), :],
                xbufs[i % 2].at[pl.ds(0, h), :],
                in_sem.at[i % 2, 0],
            ),
            pltpu.make_async_copy(
                x_hbm.at[pl.ds(i * _BR + h, r - h), :],
                xbufs[i % 2].at[pl.ds(h, r - h), :],
                in_sem.at[i % 2, 1],
            ),
        )

    def s_copy(i):
        if i < _NB - 1:
            return pltpu.make_async_copy(
                sbufs[i % 2], st_hbm.at[:, pl.ds(i * _BR, _BR)],
                os_sem.at[i % 2])
        return pltpu.make_async_copy(
            sbl, st_hbm.at[:, pl.ds(i * _BR, _R_LAST)], os_sem.at[i % 2])

    def d_copy(i):
        if i < _NB - 1:
            return pltpu.make_async_copy(
                dbufs[i % 2], dt_hbm.at[:, pl.ds(i * _BR, _BR)],
                od_sem.at[i % 2])
        return pltpu.make_async_copy(
            dbl, dt_hbm.at[:, pl.ds(i * _BR, _R_LAST)], od_sem.at[i % 2])

    bc = bc_ref[...].T
    bb = bb_ref[...].T
    wc = wc_ref[...]
    wb = wb_ref[...]

    for c in in_copies(0):
        c.start()
    for i in range(_NB):
        if i + 1 < _NB:
            for c in in_copies(i + 1):
                c.start()
        for c in in_copies(i):
            c.wait()
        if i >= 2:
            s_copy(i - 2).wait()
            d_copy(i - 2).wait()
        s = i % 2
        if i < _NB - 1:
            x = xbufs[s][...]
            sbufs[s][...] = jax.lax.dot_general(
                wc, x, _DN, preferred_element_type=jnp.float32) + bc
            dbufs[s][...] = jax.lax.dot_general(
                wb, x, _DN, preferred_element_type=jnp.float32) + bb
        else:
            x = xbufs[s][pl.ds(0, _R_LAST), :]
            sbl[...] = jax.lax.dot_general(
                wc, x, _DN, preferred_element_type=jnp.float32) + bc
            dbl[...] = jax.lax.dot_general(
                wb, x, _DN, preferred_element_type=jnp.float32) + bb
        s_copy(i).start()
        d_copy(i).start()
    for i in (_NB - 2, _NB - 1):
        s_copy(i).wait()
        d_copy(i).wait()


@jax.jit
def kernel(x, W_cls, b_cls, W_box, b_box):
    if x.ndim > 2:
        x = x.reshape(x.shape[0], -1)
    n, in_dim = x.shape
    n_cls = W_cls.shape[0]
    n_box = W_box.shape[0]

    bc = b_cls.reshape(1, n_cls)
    bb = b_box.reshape(1, n_box)

    scores_t, deltas_t = pl.pallas_call(
        _kernel,
        in_specs=[
            pl.BlockSpec(memory_space=pltpu.HBM),
            pl.BlockSpec(memory_space=pltpu.VMEM),
            pl.BlockSpec(memory_space=pltpu.VMEM),
            pl.BlockSpec(memory_space=pltpu.VMEM),
            pl.BlockSpec(memory_space=pltpu.VMEM),
        ],
        out_specs=[
            pl.BlockSpec(memory_space=pltpu.HBM),
            pl.BlockSpec(memory_space=pltpu.HBM),
        ],
        out_shape=[
            jax.ShapeDtypeStruct((n_cls, n), jnp.float32),
            jax.ShapeDtypeStruct((n_box, n), jnp.float32),
        ],
        scratch_shapes=[
            pltpu.VMEM((_BR, in_dim), jnp.float32),
            pltpu.VMEM((_BR, in_dim), jnp.float32),
            pltpu.VMEM((n_cls, _BR), jnp.float32),
            pltpu.VMEM((n_cls, _BR), jnp.float32),
            pltpu.VMEM((n_box, _BR), jnp.float32),
            pltpu.VMEM((n_box, _BR), jnp.float32),
            pltpu.VMEM((n_cls, _R_LAST), jnp.float32),
            pltpu.VMEM((n_box, _R_LAST), jnp.float32),
            pltpu.SemaphoreType.DMA((2, 2)),
            pltpu.SemaphoreType.DMA((2,)),
            pltpu.SemaphoreType.DMA((2,)),
        ],
    )(x, W_cls, bc, W_box, bb)
    return (scores_t.T, deltas_t.T)
